# baseline (speedup 1.0000x reference)
import jax
import jax.numpy as jnp
from jax.experimental import pallas as pl
from jax.experimental.pallas import tpu as pltpu

_B, _N, _D, _P = 4, 3136, 64, 8192
_H = 56
_QT = 896


def _min_dist_kernel(q_ref, ct_ref, out_ref):
    q = q_ref[...]
    ct = ct_ref[...]
    ctf = ct.astype(jnp.float32)
    cnorm = jnp.sum(ctf * ctf, axis=0, keepdims=True)
    cn_hi = cnorm.astype(jnp.bfloat16)
    cn_lo = (cnorm - cn_hi.astype(jnp.float32)).astype(jnp.bfloat16)
    b = jnp.concatenate([ct, cn_hi, cn_lo], axis=0)
    qs = -2.0 * q
    ones = jnp.ones((_QT, 1), jnp.bfloat16)
    qa = jnp.concatenate([qs, ones, ones], axis=1)
    dist = jnp.dot(qa, b, preferred_element_type=jnp.float32)
    m = jnp.min(dist, axis=1, keepdims=True)
    qf = q.astype(jnp.float32)
    qnorm = jnp.sum(qf * qf, axis=1, keepdims=True)
    out_ref[...] = jnp.sqrt(m + qnorm)


@jax.jit
def kernel(embeds, centroids, r):
    del r
    q = embeds.reshape(_B * _N, _D).astype(jnp.bfloat16)
    ct = centroids.T.astype(jnp.bfloat16)
    out = pl.pallas_call(
        _min_dist_kernel,
        grid=(_B * _N // _QT,),
        in_specs=[
            pl.BlockSpec((_QT, _D), lambda i: (i, 0)),
            pl.BlockSpec((_D, _P), lambda i: (0, 0)),
        ],
        out_specs=pl.BlockSpec((_QT, 1), lambda i: (i, 0)),
        out_shape=jax.ShapeDtypeStruct((_B * _N, 1), jnp.float32),
        compiler_params=pltpu.CompilerParams(
            dimension_semantics=("parallel",)),
    )(q, ct)
    score = jnp.transpose(out.reshape(_B, _H, _H, 1), (0, 3, 1, 2))
    return (jnp.float32(0.0), score)
